# baseline (speedup 1.0000x reference)
import functools

import jax
import jax.numpy as jnp
from jax import lax
from jax.experimental import pallas as pl
from jax.experimental.pallas import tpu as pltpu
from jax.experimental.pallas import tpu_sc as plsc

F32 = jnp.float32
I32 = jnp.int32

GSTEP = 100000.0
NROWS = 100000
KD = 128
RPW = 3136
CH = 448
NCH = RPW // CH
NGRP = CH // 16


def _tc1_body(act, pfc, w1t, b1, w2t, b2, ts, qnb_o, ma_o):
    comb = jnp.concatenate([act[...], pfc[...]], axis=1)
    h = jnp.maximum(comb @ w1t[...] + b1[...], 0.0)
    qk = h @ w2t[...] + b2[...]
    n = jnp.maximum(jnp.sqrt(jnp.sum(qk * qk)), 1e-12)
    qn = qk / n
    qnb_o[...] = jnp.broadcast_to(qn.reshape(KD, 1), (KD, 16))
    ma_o[...] = jnp.maximum(GSTEP - jnp.min(ts[...]), 1.0).reshape(1, 1)


_tc1 = pl.pallas_call(
    _tc1_body,
    out_shape=(jax.ShapeDtypeStruct((KD, 16), F32),
               jax.ShapeDtypeStruct((1, 1), F32)),
)


def _rsqrt(x):
    i = plsc.bitcast(x, I32)
    i = jnp.int32(0x5F3759DF) - (i >> 1)
    y = plsc.bitcast(i, F32)
    for _ in range(3):
        y = y * (1.5 - 0.5 * x * y * y)
    return y


def _sc_body(qn_h, keys_h, ts_h, td_h, par_h, out_h,
             qn_v, par_v, kb0, kb1, ts_v, td_v, res_v, s0, s1):
    wid = lax.axis_index("c") * 16 + lax.axis_index("s")
    base = jnp.minimum(wid * RPW, NROWS - RPW)

    pltpu.sync_copy(qn_h, qn_v)
    pltpu.sync_copy(par_h, par_v)
    pltpu.sync_copy(ts_h.at[pl.ds(base, RPW)], ts_v)
    pltpu.sync_copy(td_h.at[pl.ds(base, RPW)], td_v)

    pvs = [par_v[pl.ds(16 * i, 16)] for i in range(3)]

    def pget(m):
        return pvs[m // 16][m % 16]

    sw = [[pget(k * 3 + c) for c in range(3)] for k in range(8)]
    sb = [pget(24 + k) for k in range(8)]
    s2 = [pget(32 + k) for k in range(8)]
    sb2s = pget(40)
    maxage = pget(41)

    iot = lax.broadcasted_iota(I32, (16,), 0)
    zf = jnp.zeros((16,), F32)

    kbufs = (kb0, kb1)
    sems = (s0, s1)

    def start(c):
        return pltpu.async_copy(
            keys_h.at[pl.ds((base + c * CH) * KD, CH * KD)],
            kbufs[c % 2], sems[c % 2])

    carry = (jnp.full((16,), -3.4e38, F32), jnp.zeros((16,), I32), zf, zf)
    h = start(0)
    for c in range(NCH):
        nh = start(c + 1) if c + 1 < NCH else None
        h.wait()
        kb = kbufs[c % 2]

        def group_body(g, cr, _c=c, _kb=kb):
            br, bix, bs, bt = cr
            r0 = g * 16
            rows = r0 + iot
            fb = rows * KD

            def jc_body(jc, sc):
                s0, s1, s2, s3, n0, n1, n2, n3 = sc
                cb = jc * 16
                qb = jc * 256
                ss = [s0, s1, s2, s3]
                nn = [n0, n1, n2, n3]
                for jj in range(16):
                    x = plsc.load_gather(_kb, [fb + (cb + jj)])
                    qv = qn_v[pl.ds(qb + jj * 16, 16)]
                    ss[jj % 4] = ss[jj % 4] + x * qv
                    nn[jj % 4] = nn[jj % 4] + x * x
                return (*ss, *nn)

            s0, s1, s2, s3, n0, n1, n2, n3 = lax.fori_loop(
                0, 8, jc_body, (zf,) * 8)
            sim = (s0 + s1) + (s2 + s3)
            nrm = (n0 + n1) + (n2 + n3)
            simn = sim * _rsqrt(jnp.maximum(nrm, 1e-24))
            loc = _c * CH + r0
            ts16 = ts_v[pl.ds(loc, 16)]
            td16 = td_v[pl.ds(loc, 16)]
            tdc16 = jnp.maximum(jnp.abs(td16), 1e-6)
            rec = 1.0 - (GSTEP - ts16) / maxage
            rel = jnp.full((16,), 0.0, F32) + sb2s
            for k in range(8):
                hk = jnp.maximum(
                    sw[k][0] * simn + sw[k][1] * tdc16 + sw[k][2] * rec + sb[k],
                    0.0)
                rel = rel + s2[k] * hk
            gidx = base + _c * CH + r0 + iot
            upd = rel > br
            br = jnp.where(upd, rel, br)
            bix = jnp.where(upd, gidx, bix)
            bs = jnp.where(upd, simn, bs)
            bt = jnp.where(upd, td16, bt)
            return (br, bix, bs, bt)

        carry = lax.fori_loop(0, NGRP, group_body, carry)
        h = nh

    br, bix, bs, bt = carry
    maxv = jnp.max(br)
    bif = bix.astype(F32)
    bmin = jnp.min(jnp.where(br == maxv, bif, 3.4e38))
    sel = bif == bmin
    simw = jnp.max(jnp.where(sel, bs, -3.4e38))
    tdw = jnp.max(jnp.where(sel, bt, -3.4e38))
    res = (jnp.where(iot == 0, maxv, 0.0) + jnp.where(iot == 1, simw, 0.0)
           + jnp.where(iot == 2, tdw, 0.0) + jnp.where(iot == 3, bmin, 0.0))
    res_v[0] = res
    pltpu.sync_copy(res_v, out_h.at[pl.ds(wid, 1)])


_sc_scan = functools.partial(
    pl.kernel,
    out_type=jax.ShapeDtypeStruct((32, 16), F32),
    mesh=plsc.VectorSubcoreMesh(core_axis_name="c", subcore_axis_name="s",
                                num_cores=2, num_subcores=16),
    scratch_types=[
        pltpu.VMEM((KD * 16,), F32),
        pltpu.VMEM((48,), F32),
        pltpu.VMEM((CH * KD,), F32),
        pltpu.VMEM((CH * KD,), F32),
        pltpu.VMEM((RPW,), F32),
        pltpu.VMEM((RPW,), F32),
        pltpu.VMEM((1, 16), F32),
        pltpu.SemaphoreType.DMA,
        pltpu.SemaphoreType.DMA,
    ],
    compiler_params=pltpu.CompilerParams(needs_layout_passes=False),
)(_sc_body)


def _tc2_body(res, gw1, gb1, gw2t, gb2, rpwt, rpb, rnwt, rnb, tdc, ep_any,
              out, ep_v, sem):
    r = res[...]
    rel = r[:, 0:1]
    sim = r[:, 1:2]
    td = r[:, 2:3]
    idxf = r[:, 3:4]
    maxv = jnp.max(rel)
    eq = rel == maxv
    bif = jnp.min(jnp.where(eq, idxf, 3.4e38))
    sel = jnp.logical_and(eq, idxf == bif)
    simw = jnp.max(jnp.where(sel, sim, -3.4e38))
    tdw = jnp.max(jnp.where(sel, td, -3.4e38))
    bi = bif.astype(I32)
    copy = pltpu.make_async_copy(ep_any.at[pl.ds(bi, 1)], ep_v, sem)
    copy.start()
    copy.wait()
    ep = ep_v[...]
    tda = jnp.abs(tdc[0, 0])
    etda = jnp.abs(tdw)
    a1 = jnp.tanh(gw1[:, 0:1] * simw + gw1[:, 1:2] * tda
                  + gw1[:, 2:3] * etda + gb1[...])
    s = jnp.sum(a1 * gw2t[...]) + gb2[0, 0]
    al = jnp.tanh(jnp.full((1, 1), 0.0, F32) + s)
    pd = al * (ep @ rpwt[...] + rpb[...])
    nm = al * (ep @ rnwt[...] + rnb[...])
    out[...] = jnp.concatenate([pd, nm, jnp.zeros((1, 20), F32)], axis=1)


_tc2 = pl.pallas_call(
    _tc2_body,
    out_shape=jax.ShapeDtypeStruct((1, 64), F32),
    in_specs=[pl.BlockSpec(memory_space=pltpu.VMEM)] * 10
    + [pl.BlockSpec(memory_space=pl.ANY)],
    scratch_shapes=[pltpu.VMEM((1, 44), F32), pltpu.SemaphoreType.DMA],
)


def kernel(activation_summary, pfc_state, episodes, ep_keys, ep_td_errors,
           ep_timestamps, W1, b1, W2, b2, sW1, sb1, sW2, sb2, gW1, gb1, gW2,
           gb2, rpW, rpb, rnW, rnb, current_td_error):
    qnb, maxage = _tc1(
        activation_summary.reshape(1, KD), pfc_state.reshape(1, 32),
        W1.T, b1.reshape(1, -1), W2.T, b2.reshape(1, -1),
        ep_timestamps.reshape(8, NROWS // 8))
    par = jnp.concatenate([
        sW1.ravel(), sb1, sW2.ravel(), sb2, maxage.ravel(),
        jnp.zeros((6,), F32)])
    res = _sc_scan(qnb.reshape(KD * 16), ep_keys.reshape(NROWS * KD),
                   ep_timestamps, ep_td_errors, par)
    tdc = jnp.asarray(current_td_error, F32).reshape(1, 1)
    out = _tc2(res, gW1, gb1.reshape(16, 1), gW2.reshape(16, 1),
               gb2.reshape(1, 1), rpW.T, rpb.reshape(1, 32), rnW.T,
               rnb.reshape(1, 12), tdc, episodes)
    return out[0, :44]


# speedup vs baseline: 1.0024x; 1.0024x over previous
import functools

import jax
import jax.numpy as jnp
from jax import lax
from jax.experimental import pallas as pl
from jax.experimental.pallas import tpu as pltpu
from jax.experimental.pallas import tpu_sc as plsc

F32 = jnp.float32
I32 = jnp.int32

GSTEP = 100000.0
NROWS = 100000
KD = 128
RPW = 3136
CH = 448
NCH = RPW // CH
NGRP = CH // 16


def _tc1_body(act, pfc, w1t, b1, w2t, b2, ts, qnb_o, ma_o):
    comb = jnp.concatenate([act[...], pfc[...]], axis=1)
    h = jnp.maximum(comb @ w1t[...] + b1[...], 0.0)
    qk = h @ w2t[...] + b2[...]
    n = jnp.maximum(jnp.sqrt(jnp.sum(qk * qk)), 1e-12)
    qn = qk / n
    qnb_o[...] = jnp.broadcast_to(qn.reshape(KD, 1), (KD, 16))
    ma_o[...] = jnp.maximum(GSTEP - jnp.min(ts[...]), 1.0).reshape(1, 1)


_tc1 = pl.pallas_call(
    _tc1_body,
    out_shape=(jax.ShapeDtypeStruct((KD, 16), F32),
               jax.ShapeDtypeStruct((1, 1), F32)),
)


def _rsqrt(x):
    i = plsc.bitcast(x, I32)
    i = jnp.int32(0x5F3759DF) - (i >> 1)
    y = plsc.bitcast(i, F32)
    for _ in range(3):
        y = y * (1.5 - 0.5 * x * y * y)
    return y


def _sc_body(qn_h, keys_h, ts_h, td_h, par_h, o_rel, o_idx, o_sim, o_td,
             qn_v, par_v, kb0, kb1, ts_v, td_v, res_v, s0, s1):
    wid = lax.axis_index("c") * 16 + lax.axis_index("s")
    base = jnp.minimum(wid * RPW, NROWS - RPW)

    pltpu.sync_copy(qn_h, qn_v)
    pltpu.sync_copy(par_h, par_v)
    pltpu.sync_copy(ts_h.at[pl.ds(base, RPW)], ts_v)
    pltpu.sync_copy(td_h.at[pl.ds(base, RPW)], td_v)

    pvs = [par_v[pl.ds(16 * i, 16)] for i in range(3)]

    def pget(m):
        return pvs[m // 16][m % 16]

    sw = [[pget(k * 3 + c) for c in range(3)] for k in range(8)]
    sb = [pget(24 + k) for k in range(8)]
    s2 = [pget(32 + k) for k in range(8)]
    sb2s = pget(40)
    maxage = pget(41)

    iot = lax.broadcasted_iota(I32, (16,), 0)
    zf = jnp.zeros((16,), F32)

    kbufs = (kb0, kb1)
    sems = (s0, s1)

    def start(c):
        return pltpu.async_copy(
            keys_h.at[pl.ds((base + c * CH) * KD, CH * KD)],
            kbufs[c % 2], sems[c % 2])

    carry = (jnp.full((16,), -3.4e38, F32), jnp.zeros((16,), I32), zf, zf)
    h = start(0)
    for c in range(NCH):
        nh = start(c + 1) if c + 1 < NCH else None
        h.wait()
        kb = kbufs[c % 2]

        def group_body(g, cr, _c=c, _kb=kb):
            br, bix, bs, bt = cr
            r0 = g * 16
            rows = r0 + iot
            fb = rows * KD

            def jc_body(jc, sc):
                a0, a1, a2, a3, n0, n1, n2, n3 = sc
                cb = jc * 16
                qb = jc * 256
                ss = [a0, a1, a2, a3]
                nn = [n0, n1, n2, n3]
                for jj in range(16):
                    x = plsc.load_gather(_kb, [fb + (cb + jj)])
                    qv = qn_v[pl.ds(qb + jj * 16, 16)]
                    ss[jj % 4] = ss[jj % 4] + x * qv
                    nn[jj % 4] = nn[jj % 4] + x * x
                return (*ss, *nn)

            a0, a1, a2, a3, n0, n1, n2, n3 = lax.fori_loop(
                0, 8, jc_body, (zf,) * 8)
            sim = (a0 + a1) + (a2 + a3)
            nrm = (n0 + n1) + (n2 + n3)
            simn = sim * _rsqrt(jnp.maximum(nrm, 1e-24))
            loc = _c * CH + r0
            ts16 = ts_v[pl.ds(loc, 16)]
            td16 = td_v[pl.ds(loc, 16)]
            tdc16 = jnp.maximum(jnp.abs(td16), 1e-6)
            rec = 1.0 - (GSTEP - ts16) / maxage
            rel = jnp.full((16,), 0.0, F32) + sb2s
            for k in range(8):
                hk = jnp.maximum(
                    sw[k][0] * simn + sw[k][1] * tdc16 + sw[k][2] * rec + sb[k],
                    0.0)
                rel = rel + s2[k] * hk
            gidx = base + _c * CH + r0 + iot
            upd = rel > br
            br = jnp.where(upd, rel, br)
            bix = jnp.where(upd, gidx, bix)
            bs = jnp.where(upd, simn, bs)
            bt = jnp.where(upd, td16, bt)
            return (br, bix, bs, bt)

        carry = lax.fori_loop(0, NGRP, group_body, carry)
        h = nh

    br, bix, bs, bt = carry
    for vec, oh in ((br, o_rel), (bix.astype(F32), o_idx), (bs, o_sim),
                    (bt, o_td)):
        res_v[0] = vec
        pltpu.sync_copy(res_v, oh.at[pl.ds(wid, 1)])


_sc_scan = functools.partial(
    pl.kernel,
    out_type=(jax.ShapeDtypeStruct((32, 16), F32),) * 4,
    mesh=plsc.VectorSubcoreMesh(core_axis_name="c", subcore_axis_name="s",
                                num_cores=2, num_subcores=16),
    scratch_types=[
        pltpu.VMEM((KD * 16,), F32),
        pltpu.VMEM((48,), F32),
        pltpu.VMEM((CH * KD,), F32),
        pltpu.VMEM((CH * KD,), F32),
        pltpu.VMEM((RPW,), F32),
        pltpu.VMEM((RPW,), F32),
        pltpu.VMEM((1, 16), F32),
        pltpu.SemaphoreType.DMA,
        pltpu.SemaphoreType.DMA,
    ],
    compiler_params=pltpu.CompilerParams(needs_layout_passes=False),
)(_sc_body)


def _tc2_body(rel_r, idx_r, sim_r, td_r, gw1, gb1, gw2t, gb2, rpwt, rpb,
              rnwt, rnb, tdc, ep_any, out, ep_v, sem):
    rel = rel_r[...]
    sim = sim_r[...]
    td = td_r[...]
    idxf = idx_r[...]
    maxv = jnp.max(rel)
    eq = rel == maxv
    bif = jnp.min(jnp.where(eq, idxf, 3.4e38))
    sel = jnp.logical_and(eq, idxf == bif)
    simw = jnp.max(jnp.where(sel, sim, -3.4e38))
    tdw = jnp.max(jnp.where(sel, td, -3.4e38))
    bi = bif.astype(I32)
    copy = pltpu.make_async_copy(ep_any.at[pl.ds(bi, 1)], ep_v, sem)
    copy.start()
    copy.wait()
    ep = ep_v[...]
    tda = jnp.abs(tdc[0, 0])
    etda = jnp.abs(tdw)
    a1 = jnp.tanh(gw1[:, 0:1] * simw + gw1[:, 1:2] * tda
                  + gw1[:, 2:3] * etda + gb1[...])
    s = jnp.sum(a1 * gw2t[...]) + gb2[0, 0]
    al = jnp.tanh(jnp.full((1, 1), 0.0, F32) + s)
    pd = al * (ep @ rpwt[...] + rpb[...])
    nm = al * (ep @ rnwt[...] + rnb[...])
    out[...] = jnp.concatenate([pd, nm, jnp.zeros((1, 20), F32)], axis=1)


_tc2 = pl.pallas_call(
    _tc2_body,
    out_shape=jax.ShapeDtypeStruct((1, 64), F32),
    in_specs=[pl.BlockSpec(memory_space=pltpu.VMEM)] * 13
    + [pl.BlockSpec(memory_space=pl.ANY)],
    scratch_shapes=[pltpu.VMEM((1, 44), F32), pltpu.SemaphoreType.DMA],
)


def kernel(activation_summary, pfc_state, episodes, ep_keys, ep_td_errors,
           ep_timestamps, W1, b1, W2, b2, sW1, sb1, sW2, sb2, gW1, gb1, gW2,
           gb2, rpW, rpb, rnW, rnb, current_td_error):
    qnb, maxage = _tc1(
        activation_summary.reshape(1, KD), pfc_state.reshape(1, 32),
        W1.T, b1.reshape(1, -1), W2.T, b2.reshape(1, -1),
        ep_timestamps.reshape(8, NROWS // 8))
    par = jnp.concatenate([
        sW1.ravel(), sb1, sW2.ravel(), sb2, maxage.ravel(),
        jnp.zeros((6,), F32)])
    rel_r, idx_r, sim_r, td_r = _sc_scan(
        qnb.reshape(KD * 16), ep_keys.reshape(NROWS * KD),
        ep_timestamps, ep_td_errors, par)
    tdc = jnp.asarray(current_td_error, F32).reshape(1, 1)
    out = _tc2(rel_r, idx_r, sim_r, td_r, gW1, gb1.reshape(16, 1),
               gW2.reshape(16, 1), gb2.reshape(1, 1), rpW.T,
               rpb.reshape(1, 32), rnW.T, rnb.reshape(1, 12), tdc, episodes)
    return out[0, :44]


# speedup vs baseline: 1.4561x; 1.4526x over previous
import functools

import jax
import jax.numpy as jnp
from jax import lax
from jax.experimental import pallas as pl
from jax.experimental.pallas import tpu as pltpu
from jax.experimental.pallas import tpu_sc as plsc

F32 = jnp.float32
I32 = jnp.int32

GSTEP = 100000.0
NROWS = 100000
KD = 128
RPW = 3136
CH = 224
NCH = RPW // CH
NGRP = CH // 16


def _tc1_body(act, pfc, w1t, b1, w2t, b2, ts, qnb_o, ma_o):
    comb = jnp.concatenate([act[...], pfc[...]], axis=1)
    h = jnp.maximum(comb @ w1t[...] + b1[...], 0.0)
    qk = h @ w2t[...] + b2[...]
    n = jnp.maximum(jnp.sqrt(jnp.sum(qk * qk)), 1e-12)
    qnb_o[...] = qk / n
    ma_o[...] = jnp.maximum(GSTEP - jnp.min(ts[...]), 1.0).reshape(1, 1)


_tc1 = pl.pallas_call(
    _tc1_body,
    out_shape=(jax.ShapeDtypeStruct((1, KD), F32),
               jax.ShapeDtypeStruct((1, 1), F32)),
)


def _rsqrt(x):
    i = plsc.bitcast(x, I32)
    i = jnp.int32(0x5F3759DF) - (i >> 1)
    y = plsc.bitcast(i, F32)
    for _ in range(3):
        y = y * (1.5 - 0.5 * x * y * y)
    return y


def _sc_body(qn_h, keys_h, ts_h, td_h, par_h, o_rel, o_idx, o_sim, o_td,
             qn_v, par_v, kb0, kb1, ts_v, td_v, res_v, s0, s1):
    wid = lax.axis_index("c") * 16 + lax.axis_index("s")
    base = jnp.minimum(wid * RPW, NROWS - RPW)

    pltpu.sync_copy(qn_h, qn_v)
    pltpu.sync_copy(par_h, par_v)
    pltpu.sync_copy(ts_h.at[pl.ds(base, RPW)], ts_v)
    pltpu.sync_copy(td_h.at[pl.ds(base, RPW)], td_v)

    pvs = [par_v[pl.ds(16 * i, 16)] for i in range(3)]

    def pget(m):
        return pvs[m // 16][m % 16]

    sw = [[pget(k * 3 + c) for c in range(3)] for k in range(8)]
    sb = [pget(24 + k) for k in range(8)]
    s2 = [pget(32 + k) for k in range(8)]
    sb2s = pget(40)
    maxage = pget(41)

    iot = lax.broadcasted_iota(I32, (16,), 0)
    zf = jnp.zeros((16,), F32)
    qc = [qn_v[pl.ds(16 * i, 16)] for i in range(8)]

    def dma(c, kb, sem):
        return pltpu.make_async_copy(
            keys_h.at[pl.ds((base + c * CH) * KD, CH * KD)], kb, sem)

    dma(0, kb0, s0).start()
    init = (jnp.full((16,), -3.4e38, F32), jnp.zeros((16,), I32), zf, zf)

    def chunk_body(c, cr):
        def proc(kb, sem_w, kb_n, sem_n):
            dma(c, kb, sem_w).wait()

            @pl.when(c + 1 < NCH)
            def _prefetch():
                dma(c + 1, kb_n, sem_n).start()

            def group_body(g, gcr):
                br, bix, bs, bt = gcr
                r0 = g * 16
                sdv = zf
                nmv = zf
                for r in range(16):
                    rb = (r0 + r) * KD
                    xs = [kb[pl.ds(rb + 16 * cc, 16)] for cc in range(8)]
                    sA = ((xs[0] * qc[0] + xs[1] * qc[1])
                          + (xs[2] * qc[2] + xs[3] * qc[3]))
                    sB = ((xs[4] * qc[4] + xs[5] * qc[5])
                          + (xs[6] * qc[6] + xs[7] * qc[7]))
                    nA = ((xs[0] * xs[0] + xs[1] * xs[1])
                          + (xs[2] * xs[2] + xs[3] * xs[3]))
                    nB = ((xs[4] * xs[4] + xs[5] * xs[5])
                          + (xs[6] * xs[6] + xs[7] * xs[7]))
                    sd = jnp.sum(sA + sB)
                    nm = jnp.sum(nA + nB)
                    lane = iot == r
                    sdv = sdv + jnp.where(lane, sd, 0.0)
                    nmv = nmv + jnp.where(lane, nm, 0.0)
                simn = sdv * _rsqrt(jnp.maximum(nmv, 1e-24))
                loc = c * CH + r0
                ts16 = ts_v[pl.ds(loc, 16)]
                td16 = td_v[pl.ds(loc, 16)]
                tdc16 = jnp.maximum(jnp.abs(td16), 1e-6)
                rec = 1.0 - (GSTEP - ts16) / maxage
                rel = jnp.full((16,), 0.0, F32) + sb2s
                for k in range(8):
                    hk = jnp.maximum(
                        sw[k][0] * simn + sw[k][1] * tdc16 + sw[k][2] * rec
                        + sb[k], 0.0)
                    rel = rel + s2[k] * hk
                gidx = base + c * CH + r0 + iot
                upd = rel > br
                br = jnp.where(upd, rel, br)
                bix = jnp.where(upd, gidx, bix)
                bs = jnp.where(upd, simn, bs)
                bt = jnp.where(upd, td16, bt)
                return (br, bix, bs, bt)

            return lax.fori_loop(0, NGRP, group_body, cr)

        return lax.cond(c % 2 == 0,
                        lambda: proc(kb0, s0, kb1, s1),
                        lambda: proc(kb1, s1, kb0, s0))

    br, bix, bs, bt = lax.fori_loop(0, NCH, chunk_body, init)
    for vec, oh in ((br, o_rel), (bix.astype(F32), o_idx), (bs, o_sim),
                    (bt, o_td)):
        res_v[0] = vec
        pltpu.sync_copy(res_v, oh.at[pl.ds(wid, 1)])


_sc_scan = functools.partial(
    pl.kernel,
    out_type=(jax.ShapeDtypeStruct((32, 16), F32),) * 4,
    mesh=plsc.VectorSubcoreMesh(core_axis_name="c", subcore_axis_name="s",
                                num_cores=2, num_subcores=16),
    scratch_types=[
        pltpu.VMEM((KD,), F32),
        pltpu.VMEM((48,), F32),
        pltpu.VMEM((CH * KD,), F32),
        pltpu.VMEM((CH * KD,), F32),
        pltpu.VMEM((RPW,), F32),
        pltpu.VMEM((RPW,), F32),
        pltpu.VMEM((1, 16), F32),
        pltpu.SemaphoreType.DMA,
        pltpu.SemaphoreType.DMA,
    ],
    compiler_params=pltpu.CompilerParams(needs_layout_passes=False),
)(_sc_body)


def _tc2_body(rel_r, idx_r, sim_r, td_r, gw1, gb1, gw2t, gb2, rpwt, rpb,
              rnwt, rnb, tdc, ep_any, out, ep_v, sem):
    rel = rel_r[...]
    sim = sim_r[...]
    td = td_r[...]
    idxf = idx_r[...]
    maxv = jnp.max(rel)
    eq = rel == maxv
    bif = jnp.min(jnp.where(eq, idxf, 3.4e38))
    sel = jnp.logical_and(eq, idxf == bif)
    simw = jnp.max(jnp.where(sel, sim, -3.4e38))
    tdw = jnp.max(jnp.where(sel, td, -3.4e38))
    bi = bif.astype(I32)
    copy = pltpu.make_async_copy(ep_any.at[pl.ds(bi, 1)], ep_v, sem)
    copy.start()
    copy.wait()
    ep = ep_v[...]
    tda = jnp.abs(tdc[0, 0])
    etda = jnp.abs(tdw)
    a1 = jnp.tanh(gw1[:, 0:1] * simw + gw1[:, 1:2] * tda
                  + gw1[:, 2:3] * etda + gb1[...])
    s = jnp.sum(a1 * gw2t[...]) + gb2[0, 0]
    al = jnp.tanh(jnp.full((1, 1), 0.0, F32) + s)
    pd = al * (ep @ rpwt[...] + rpb[...])
    nm = al * (ep @ rnwt[...] + rnb[...])
    out[...] = jnp.concatenate([pd, nm, jnp.zeros((1, 20), F32)], axis=1)


_tc2 = pl.pallas_call(
    _tc2_body,
    out_shape=jax.ShapeDtypeStruct((1, 64), F32),
    in_specs=[pl.BlockSpec(memory_space=pltpu.VMEM)] * 13
    + [pl.BlockSpec(memory_space=pl.ANY)],
    scratch_shapes=[pltpu.VMEM((1, 44), F32), pltpu.SemaphoreType.DMA],
)


def kernel(activation_summary, pfc_state, episodes, ep_keys, ep_td_errors,
           ep_timestamps, W1, b1, W2, b2, sW1, sb1, sW2, sb2, gW1, gb1, gW2,
           gb2, rpW, rpb, rnW, rnb, current_td_error):
    qnb, maxage = _tc1(
        activation_summary.reshape(1, KD), pfc_state.reshape(1, 32),
        W1.T, b1.reshape(1, -1), W2.T, b2.reshape(1, -1),
        ep_timestamps.reshape(8, NROWS // 8))
    par = jnp.concatenate([
        sW1.ravel(), sb1, sW2.ravel(), sb2, maxage.ravel(),
        jnp.zeros((6,), F32)])
    rel_r, idx_r, sim_r, td_r = _sc_scan(
        qnb.reshape(KD), ep_keys.reshape(NROWS * KD),
        ep_timestamps, ep_td_errors, par)
    tdc = jnp.asarray(current_td_error, F32).reshape(1, 1)
    out = _tc2(rel_r, idx_r, sim_r, td_r, gW1, gb1.reshape(16, 1),
               gW2.reshape(16, 1), gb2.reshape(1, 1), rpW.T,
               rpb.reshape(1, 32), rnW.T, rnb.reshape(1, 12), tdc, episodes)
    return out[0, :44]


# speedup vs baseline: 2.5154x; 1.7275x over previous
import functools

import jax
import jax.numpy as jnp
from jax import lax
from jax.experimental import pallas as pl
from jax.experimental.pallas import tpu as pltpu
from jax.experimental.pallas import tpu_sc as plsc

F32 = jnp.float32
I32 = jnp.int32

GSTEP = 100000.0
NROWS = 100000
KD = 128
RPW = 3136
CH = 224
NCH = RPW // CH
NGRP = CH // 16


def _tc1_body(act, pfc, w1t, b1, w2t, b2, ts, qnb_o, ma_o):
    comb = jnp.concatenate([act[...], pfc[...]], axis=1)
    h = jnp.maximum(comb @ w1t[...] + b1[...], 0.0)
    qk = h @ w2t[...] + b2[...]
    n = jnp.maximum(jnp.sqrt(jnp.sum(qk * qk)), 1e-12)
    qnb_o[...] = qk / n
    ma_o[...] = jnp.maximum(GSTEP - jnp.min(ts[...]), 1.0).reshape(1, 1)


_tc1 = pl.pallas_call(
    _tc1_body,
    out_shape=(jax.ShapeDtypeStruct((1, KD), F32),
               jax.ShapeDtypeStruct((1, 1), F32)),
)


def _rsqrt(x):
    i = plsc.bitcast(x, I32)
    i = jnp.int32(0x5F3759DF) - (i >> 1)
    y = plsc.bitcast(i, F32)
    for _ in range(3):
        y = y * (1.5 - 0.5 * x * y * y)
    return y


def _sc_body(qn_h, keys_h, ts_h, td_h, par_h, o_rel, o_idx, o_sim, o_td,
             qn_v, par_v, kb0, kb1, ts_v, td_v, res_v, pb_v, nb_v, s0, s1):
    wid = lax.axis_index("c") * 16 + lax.axis_index("s")
    base = jnp.minimum(wid * RPW, NROWS - RPW)

    pltpu.sync_copy(qn_h, qn_v)
    pltpu.sync_copy(par_h, par_v)
    pltpu.sync_copy(ts_h.at[pl.ds(base, RPW)], ts_v)
    pltpu.sync_copy(td_h.at[pl.ds(base, RPW)], td_v)

    pvs = [par_v[pl.ds(16 * i, 16)] for i in range(3)]

    def pget(m):
        return pvs[m // 16][m % 16]

    sw = [[pget(k * 3 + c) for c in range(3)] for k in range(8)]
    sb = [pget(24 + k) for k in range(8)]
    s2 = [pget(32 + k) for k in range(8)]
    sb2s = pget(40)
    maxage = pget(41)

    iot = lax.broadcasted_iota(I32, (16,), 0)
    zf = jnp.zeros((16,), F32)
    qc = [qn_v[pl.ds(16 * i, 16)] for i in range(8)]

    def dma(c, kb, sem):
        return pltpu.make_async_copy(
            keys_h.at[pl.ds((base + c * CH) * KD, CH * KD)], kb, sem)

    dma(0, kb0, s0).start()
    init = (jnp.full((16,), -3.4e38, F32), jnp.zeros((16,), I32), zf, zf, *qc)

    def chunk_body(c, cr):
        def proc(kb, sem_w, kb_n, sem_n):
            dma(c, kb, sem_w).wait()

            @pl.when(c + 1 < NCH)
            def _prefetch():
                dma(c + 1, kb_n, sem_n).start()

            def group_body(g, gcr):
                br, bix, bs, bt, *qr = gcr
                r0 = g * 16
                for r in range(16):
                    rb = (r0 + r) * KD
                    xs = [kb[pl.ds(rb + 16 * cc, 16)] for cc in range(8)]
                    sA = ((xs[0] * qr[0] + xs[1] * qr[1])
                          + (xs[2] * qr[2] + xs[3] * qr[3]))
                    sB = ((xs[4] * qr[4] + xs[5] * qr[5])
                          + (xs[6] * qr[6] + xs[7] * qr[7]))
                    nA = ((xs[0] * xs[0] + xs[1] * xs[1])
                          + (xs[2] * xs[2] + xs[3] * xs[3]))
                    nB = ((xs[4] * xs[4] + xs[5] * xs[5])
                          + (xs[6] * xs[6] + xs[7] * xs[7]))
                    pb_v[pl.ds(r * 17, 16)] = sA + sB
                    nb_v[pl.ds(r * 17, 16)] = nA + nB
                tix = iot * 17
                sacc = [None] * 4
                nacc = [None] * 4
                for j in range(16):
                    tj = plsc.load_gather(pb_v, [tix + j])
                    uj = plsc.load_gather(nb_v, [tix + j])
                    k4 = j % 4
                    sacc[k4] = tj if sacc[k4] is None else sacc[k4] + tj
                    nacc[k4] = uj if nacc[k4] is None else nacc[k4] + uj
                sdv = (sacc[0] + sacc[1]) + (sacc[2] + sacc[3])
                nmv = (nacc[0] + nacc[1]) + (nacc[2] + nacc[3])
                simn = sdv * _rsqrt(jnp.maximum(nmv, 1e-24))
                loc = c * CH + r0
                ts16 = ts_v[pl.ds(loc, 16)]
                td16 = td_v[pl.ds(loc, 16)]
                tdc16 = jnp.maximum(jnp.abs(td16), 1e-6)
                rec = 1.0 - (GSTEP - ts16) / maxage
                rel = jnp.full((16,), 0.0, F32) + sb2s
                for k in range(8):
                    hk = jnp.maximum(
                        sw[k][0] * simn + sw[k][1] * tdc16 + sw[k][2] * rec
                        + sb[k], 0.0)
                    rel = rel + s2[k] * hk
                gidx = base + c * CH + r0 + iot
                upd = rel > br
                br = jnp.where(upd, rel, br)
                bix = jnp.where(upd, gidx, bix)
                bs = jnp.where(upd, simn, bs)
                bt = jnp.where(upd, td16, bt)
                return (br, bix, bs, bt, *qr)

            return lax.fori_loop(0, NGRP, group_body, cr)

        return lax.cond(c % 2 == 0,
                        lambda: proc(kb0, s0, kb1, s1),
                        lambda: proc(kb1, s1, kb0, s0))

    br, bix, bs, bt, *_qf = lax.fori_loop(0, NCH, chunk_body, init)
    for vec, oh in ((br, o_rel), (bix.astype(F32), o_idx), (bs, o_sim),
                    (bt, o_td)):
        res_v[0] = vec
        pltpu.sync_copy(res_v, oh.at[pl.ds(wid, 1)])


_sc_scan = functools.partial(
    pl.kernel,
    out_type=(jax.ShapeDtypeStruct((32, 16), F32),) * 4,
    mesh=plsc.VectorSubcoreMesh(core_axis_name="c", subcore_axis_name="s",
                                num_cores=2, num_subcores=16),
    scratch_types=[
        pltpu.VMEM((KD,), F32),
        pltpu.VMEM((48,), F32),
        pltpu.VMEM((CH * KD,), F32),
        pltpu.VMEM((CH * KD,), F32),
        pltpu.VMEM((RPW,), F32),
        pltpu.VMEM((RPW,), F32),
        pltpu.VMEM((1, 16), F32),
        pltpu.VMEM((272,), F32),
        pltpu.VMEM((272,), F32),
        pltpu.SemaphoreType.DMA,
        pltpu.SemaphoreType.DMA,
    ],
    compiler_params=pltpu.CompilerParams(needs_layout_passes=False),
)(_sc_body)


def _tc2_body(rel_r, idx_r, sim_r, td_r, gw1, gb1, gw2t, gb2, rpwt, rpb,
              rnwt, rnb, tdc, ep_any, out, ep_v, sem):
    rel = rel_r[...]
    sim = sim_r[...]
    td = td_r[...]
    idxf = idx_r[...]
    maxv = jnp.max(rel)
    eq = rel == maxv
    bif = jnp.min(jnp.where(eq, idxf, 3.4e38))
    sel = jnp.logical_and(eq, idxf == bif)
    simw = jnp.max(jnp.where(sel, sim, -3.4e38))
    tdw = jnp.max(jnp.where(sel, td, -3.4e38))
    bi = bif.astype(I32)
    copy = pltpu.make_async_copy(ep_any.at[pl.ds(bi, 1)], ep_v, sem)
    copy.start()
    copy.wait()
    ep = ep_v[...]
    tda = jnp.abs(tdc[0, 0])
    etda = jnp.abs(tdw)
    a1 = jnp.tanh(gw1[:, 0:1] * simw + gw1[:, 1:2] * tda
                  + gw1[:, 2:3] * etda + gb1[...])
    s = jnp.sum(a1 * gw2t[...]) + gb2[0, 0]
    al = jnp.tanh(jnp.full((1, 1), 0.0, F32) + s)
    pd = al * (ep @ rpwt[...] + rpb[...])
    nm = al * (ep @ rnwt[...] + rnb[...])
    out[...] = jnp.concatenate([pd, nm, jnp.zeros((1, 20), F32)], axis=1)


_tc2 = pl.pallas_call(
    _tc2_body,
    out_shape=jax.ShapeDtypeStruct((1, 64), F32),
    in_specs=[pl.BlockSpec(memory_space=pltpu.VMEM)] * 13
    + [pl.BlockSpec(memory_space=pl.ANY)],
    scratch_shapes=[pltpu.VMEM((1, 44), F32), pltpu.SemaphoreType.DMA],
)


def kernel(activation_summary, pfc_state, episodes, ep_keys, ep_td_errors,
           ep_timestamps, W1, b1, W2, b2, sW1, sb1, sW2, sb2, gW1, gb1, gW2,
           gb2, rpW, rpb, rnW, rnb, current_td_error):
    qnb, maxage = _tc1(
        activation_summary.reshape(1, KD), pfc_state.reshape(1, 32),
        W1.T, b1.reshape(1, -1), W2.T, b2.reshape(1, -1),
        ep_timestamps.reshape(8, NROWS // 8))
    par = jnp.concatenate([
        sW1.ravel(), sb1, sW2.ravel(), sb2, maxage.ravel(),
        jnp.zeros((6,), F32)])
    rel_r, idx_r, sim_r, td_r = _sc_scan(
        qnb.reshape(KD), ep_keys.reshape(NROWS * KD),
        ep_timestamps, ep_td_errors, par)
    tdc = jnp.asarray(current_td_error, F32).reshape(1, 1)
    out = _tc2(rel_r, idx_r, sim_r, td_r, gW1, gb1.reshape(16, 1),
               gW2.reshape(16, 1), gb2.reshape(1, 1), rpW.T,
               rpb.reshape(1, 32), rnW.T, rnb.reshape(1, 12), tdc, episodes)
    return out[0, :44]
